# Initial kernel scaffold; baseline (speedup 1.0000x reference)
import jax
import jax.numpy as jnp
from jax.experimental import pallas as pl


def kernel(variables, factors, edge_index, edge_attr, batch_idx, Wm_vf, bm_vf, Wc_vf, bc_vf, Wm_fv, bm_fv, Wc_fv, bc_fv):
    raise NotImplementedError("write your pallas kernel here")


# speedup vs baseline: 4.7119x; 4.7119x over previous
import functools

import jax
import jax.numpy as jnp
from jax import lax
from jax.experimental import pallas as pl
from jax.experimental.pallas import tpu as pltpu
from jax.experimental.pallas import tpu_sc as plsc

EMBED = 128
N_NODE = 10000
N_EDGE = 320000

NUM_CORES = 2
NUM_SUBCORES = 16
NUM_WORKERS = NUM_CORES * NUM_SUBCORES
EDGES_PER_WORKER = N_EDGE // NUM_WORKERS
E_BLK = 80
N_BLKS = EDGES_PER_WORKER // E_BLK
ROWS_MAIN = 624
ROWS_LAST = 640

_TC_ROWS = 2000
_TC_GRID = N_NODE // _TC_ROWS


def _tc_pre_body(f_ref, v_ref, w1_ref, w2_ref, bm_ref, a_ref, b_ref):
    a_ref[...] = jnp.dot(f_ref[...], w1_ref[...],
                         preferred_element_type=jnp.float32) + bm_ref[...]
    b_ref[...] = jnp.dot(v_ref[...], w2_ref[...],
                         preferred_element_type=jnp.float32)


def _tc_mid_body(f_ref, v_ref, p_ref, wc1_ref, wc2_ref, bc_ref,
                 w3_ref, w4_ref, bm2_ref, nf_ref, c_ref, d_ref):
    aggr = p_ref[0] + p_ref[1]
    nf = jax.nn.relu(
        jnp.dot(f_ref[...], wc1_ref[...], preferred_element_type=jnp.float32)
        + jnp.dot(aggr, wc2_ref[...], preferred_element_type=jnp.float32)
        + bc_ref[...])
    nf_ref[...] = nf
    c_ref[...] = jnp.dot(v_ref[...], w3_ref[...],
                         preferred_element_type=jnp.float32)
    d_ref[...] = jnp.dot(nf, w4_ref[...],
                         preferred_element_type=jnp.float32) + bm2_ref[...]


def _tc_post_body(v_ref, q_ref, wc3_ref, wc4_ref, bc2_ref, nv_ref):
    aggr = q_ref[0] + q_ref[1]
    nv_ref[...] = v_ref[...] + jax.nn.relu(
        jnp.dot(v_ref[...], wc3_ref[...], preferred_element_type=jnp.float32)
        + jnp.dot(aggr, wc4_ref[...], preferred_element_type=jnp.float32)
        + bc2_ref[...])


def _row_spec():
    return pl.BlockSpec((_TC_ROWS, EMBED), lambda i: (i, 0))


def _full_spec(shape):
    n = len(shape)
    return pl.BlockSpec(shape, lambda i: (0,) * n)


def _part_spec():
    return pl.BlockSpec((NUM_CORES, _TC_ROWS, EMBED), lambda i: (0, i, 0))


_mat = functools.partial(jax.ShapeDtypeStruct, dtype=jnp.float32)


def _tc_pre(factors, variables, w1, w2, bm):
    return pl.pallas_call(
        _tc_pre_body,
        grid=(_TC_GRID,),
        in_specs=[_row_spec(), _row_spec(), _full_spec((EMBED, EMBED)),
                  _full_spec((EMBED, EMBED)), _full_spec((1, EMBED))],
        out_specs=[_row_spec(), _row_spec()],
        out_shape=[_mat((N_NODE, EMBED)), _mat((N_NODE, EMBED))],
    )(factors, variables, w1, w2, bm)


def _tc_mid(factors, variables, part, wc1, wc2, bc, w3, w4, bm2):
    return pl.pallas_call(
        _tc_mid_body,
        grid=(_TC_GRID,),
        in_specs=[_row_spec(), _row_spec(), _part_spec(),
                  _full_spec((EMBED, EMBED)), _full_spec((EMBED, EMBED)),
                  _full_spec((1, EMBED)), _full_spec((EMBED, EMBED)),
                  _full_spec((EMBED, EMBED)), _full_spec((1, EMBED))],
        out_specs=[_row_spec(), _row_spec(), _row_spec()],
        out_shape=[_mat((N_NODE, EMBED)), _mat((N_NODE, EMBED)),
                   _mat((N_NODE, EMBED))],
    )(factors, variables, part, wc1, wc2, bc, w3, w4, bm2)


def _tc_post(variables, part, wc3, wc4, bc2):
    return pl.pallas_call(
        _tc_post_body,
        grid=(_TC_GRID,),
        in_specs=[_row_spec(), _part_spec(), _full_spec((EMBED, EMBED)),
                  _full_spec((EMBED, EMBED)), _full_spec((1, EMBED))],
        out_specs=_row_spec(),
        out_shape=_mat((N_NODE, EMBED)),
    )(variables, part, wc3, wc4, bc2)


def _sc_edge_body(g1_hbm, i1_hbm, g2_hbm, i2_hbm, zero_hbm, out_hbm,
                  i1_v, i2_v, buf_a, buf_b, aggr_sh, sem_a, sem_b):
    c = lax.axis_index("c")
    s = lax.axis_index("s")
    wid = c * NUM_SUBCORES + s
    row_off = pl.multiple_of(s * ROWS_MAIN, 8)

    @pl.when(s < NUM_SUBCORES - 1)
    def _zero_main():
        pltpu.sync_copy(zero_hbm.at[pl.ds(0, ROWS_MAIN)],
                        aggr_sh.at[pl.ds(row_off, ROWS_MAIN)])

    @pl.when(s == NUM_SUBCORES - 1)
    def _zero_last():
        pltpu.sync_copy(zero_hbm, aggr_sh.at[pl.ds(row_off, ROWS_LAST)])

    plsc.subcore_barrier()

    base = wid * EDGES_PER_WORKER

    @pl.loop(0, N_BLKS)
    def _edge_block(i):
        off = base + i * E_BLK
        pltpu.sync_copy(i1_hbm.at[pl.ds(off, E_BLK)], i1_v)
        pltpu.sync_copy(i2_hbm.at[pl.ds(off, E_BLK)], i2_v)
        cp_a = pltpu.async_copy(g1_hbm.at[i1_v], buf_a, sem_a)
        cp_b = pltpu.async_copy(g2_hbm.at[i2_v], buf_b, sem_b)
        cp_a.wait()
        cp_b.wait()

        @pl.loop(0, E_BLK)
        def _relu_row(r):
            for cb in range(EMBED // 16):
                sl = pl.ds(cb * 16, 16)
                buf_a[r, sl] = jnp.maximum(buf_a[r, sl] + buf_b[r, sl], 0.0)

        pltpu.sync_copy(buf_a, aggr_sh.at[i2_v], add=True)

    plsc.subcore_barrier()

    @pl.when(s < NUM_SUBCORES - 1)
    def _out_main():
        rows = pl.ds(row_off, ROWS_MAIN)
        pltpu.sync_copy(aggr_sh.at[rows], out_hbm.at[c, rows])

    @pl.when(s == NUM_SUBCORES - 1)
    def _out_last():
        rows = pl.ds(row_off, ROWS_LAST)
        pltpu.sync_copy(aggr_sh.at[rows], out_hbm.at[c, rows])


@functools.lru_cache(maxsize=None)
def _make_sc_edge_pass():
    return pl.kernel(
        _sc_edge_body,
        out_type=jax.ShapeDtypeStruct((NUM_CORES, N_NODE, EMBED),
                                      jnp.float32),
        mesh=plsc.VectorSubcoreMesh(core_axis_name="c",
                                    subcore_axis_name="s"),
        scratch_types=[
            pltpu.VMEM((E_BLK,), jnp.int32),
            pltpu.VMEM((E_BLK,), jnp.int32),
            pltpu.VMEM((E_BLK, EMBED), jnp.float32),
            pltpu.VMEM((E_BLK, EMBED), jnp.float32),
            pltpu.VMEM_SHARED((N_NODE, EMBED), jnp.float32),
            pltpu.SemaphoreType.DMA,
            pltpu.SemaphoreType.DMA,
        ],
    )


def kernel(variables, factors, edge_index, edge_attr, batch_idx,
           Wm_vf, bm_vf, Wc_vf, bc_vf, Wm_fv, bm_fv, Wc_fv, bc_fv):
    del edge_attr, batch_idx
    src = edge_index[0].astype(jnp.int32)
    dst = edge_index[1].astype(jnp.int32)
    zeros = jnp.zeros((ROWS_LAST, EMBED), jnp.float32)

    def half(w):
        return w[:EMBED], w[EMBED:]

    w1, w2 = half(Wm_vf)
    wc1, wc2 = half(Wc_vf)
    w3, w4 = half(Wm_fv)
    wc3, wc4 = half(Wc_fv)

    bm = bm_vf.reshape(1, EMBED)
    bc = bc_vf.reshape(1, EMBED)
    bm2 = bm_fv.reshape(1, EMBED)
    bc2 = bc_fv.reshape(1, EMBED)

    sc_edge_pass = _make_sc_edge_pass()

    a_proj, b_proj = _tc_pre(factors, variables, w1, w2, bm)
    part1 = sc_edge_pass(b_proj, src, a_proj, dst, zeros)
    new_factors, c_proj, d_proj = _tc_mid(
        factors, variables, part1, wc1, wc2, bc, w3, w4, bm2)

    part2 = sc_edge_pass(d_proj, dst, c_proj, src, zeros)
    new_variables = _tc_post(variables, part2, wc3, wc4, bc2)

    return new_variables, new_factors


# speedup vs baseline: 8.8421x; 1.8765x over previous
import functools

import jax
import jax.numpy as jnp
from jax import lax
from jax.experimental import pallas as pl
from jax.experimental.pallas import tpu as pltpu
from jax.experimental.pallas import tpu_sc as plsc

EMBED = 128
HALF = EMBED // 2
N_NODE = 10000
N_EDGE = 320000

NUM_CORES = 2
NUM_SUBCORES = 16
EDGES_PER_TILE = N_EDGE // NUM_SUBCORES
E_BLK = 80
N_BLKS = EDGES_PER_TILE // E_BLK

ROWS_MAIN = 624
ROWS_LAST = 640

_TC_ROWS = 2000
_TC_GRID = N_NODE // _TC_ROWS


def _split(x):
    return x[:, :HALF], x[:, HALF:]


def _tc_pre_body(f_ref, v_ref, w1_ref, w2_ref, bm_ref, a_ref, b_ref):
    a = jnp.dot(f_ref[...], w1_ref[...],
                preferred_element_type=jnp.float32) + bm_ref[...]
    b = jnp.dot(v_ref[...], w2_ref[...], preferred_element_type=jnp.float32)
    a_ref[0], a_ref[1] = _split(a)
    b_ref[0], b_ref[1] = _split(b)


def _tc_mid_body(f_ref, v_ref, p_ref, wc1_ref, wc2_ref, bc_ref,
                 w3_ref, w4_ref, bm2_ref, nf_ref, c_ref, d_ref):
    aggr = jnp.concatenate([p_ref[0], p_ref[1]], axis=-1)
    nf = jax.nn.relu(
        jnp.dot(f_ref[...], wc1_ref[...], preferred_element_type=jnp.float32)
        + jnp.dot(aggr, wc2_ref[...], preferred_element_type=jnp.float32)
        + bc_ref[...])
    nf_ref[...] = nf
    c = jnp.dot(v_ref[...], w3_ref[...], preferred_element_type=jnp.float32)
    d = jnp.dot(nf, w4_ref[...],
                preferred_element_type=jnp.float32) + bm2_ref[...]
    c_ref[0], c_ref[1] = _split(c)
    d_ref[0], d_ref[1] = _split(d)


def _tc_post_body(v_ref, q_ref, wc3_ref, wc4_ref, bc2_ref, nv_ref):
    aggr = jnp.concatenate([q_ref[0], q_ref[1]], axis=-1)
    nv_ref[...] = v_ref[...] + jax.nn.relu(
        jnp.dot(v_ref[...], wc3_ref[...], preferred_element_type=jnp.float32)
        + jnp.dot(aggr, wc4_ref[...], preferred_element_type=jnp.float32)
        + bc2_ref[...])


def _row_spec():
    return pl.BlockSpec((_TC_ROWS, EMBED), lambda i: (i, 0))


def _full_spec(shape):
    n = len(shape)
    return pl.BlockSpec(shape, lambda i: (0,) * n)


def _split_spec():
    return pl.BlockSpec((NUM_CORES, _TC_ROWS, HALF), lambda i: (0, i, 0))


_mat = functools.partial(jax.ShapeDtypeStruct, dtype=jnp.float32)
_SPLIT_SHAPE = (NUM_CORES, N_NODE, HALF)


def _tc_pre(factors, variables, w1, w2, bm):
    return pl.pallas_call(
        _tc_pre_body,
        grid=(_TC_GRID,),
        in_specs=[_row_spec(), _row_spec(), _full_spec((EMBED, EMBED)),
                  _full_spec((EMBED, EMBED)), _full_spec((1, EMBED))],
        out_specs=[_split_spec(), _split_spec()],
        out_shape=[_mat(_SPLIT_SHAPE), _mat(_SPLIT_SHAPE)],
    )(factors, variables, w1, w2, bm)


def _tc_mid(factors, variables, part, wc1, wc2, bc, w3, w4, bm2):
    return pl.pallas_call(
        _tc_mid_body,
        grid=(_TC_GRID,),
        in_specs=[_row_spec(), _row_spec(), _split_spec(),
                  _full_spec((EMBED, EMBED)), _full_spec((EMBED, EMBED)),
                  _full_spec((1, EMBED)), _full_spec((EMBED, EMBED)),
                  _full_spec((EMBED, EMBED)), _full_spec((1, EMBED))],
        out_specs=[_row_spec(), _split_spec(), _split_spec()],
        out_shape=[_mat((N_NODE, EMBED)), _mat(_SPLIT_SHAPE),
                   _mat(_SPLIT_SHAPE)],
    )(factors, variables, part, wc1, wc2, bc, w3, w4, bm2)


def _tc_post(variables, part, wc3, wc4, bc2):
    return pl.pallas_call(
        _tc_post_body,
        grid=(_TC_GRID,),
        in_specs=[_row_spec(), _split_spec(), _full_spec((EMBED, EMBED)),
                  _full_spec((EMBED, EMBED)), _full_spec((1, EMBED))],
        out_specs=_row_spec(),
        out_shape=_mat((N_NODE, EMBED)),
    )(variables, part, wc3, wc4, bc2)


def _sc_edge_body(g1_hbm, i1_hbm, g2_hbm, i2_hbm, zero_hbm, out_hbm,
                  i1_all, i2_all, a0, a1, b0, b1, s0, s1, aggr_sh,
                  sga0, sga1, sgb0, sgb1, ssc0, ssc1):
    buf_a, buf_b, buf_s = (a0, a1), (b0, b1), (s0, s1)
    sem_ga, sem_gb, sem_sc = (sga0, sga1), (sgb0, sgb1), (ssc0, ssc1)
    c = lax.axis_index("c")
    s = lax.axis_index("s")
    row_off = pl.multiple_of(s * ROWS_MAIN, 8)

    @pl.when(s < NUM_SUBCORES - 1)
    def _zero_main():
        pltpu.sync_copy(zero_hbm.at[pl.ds(0, ROWS_MAIN)],
                        aggr_sh.at[pl.ds(row_off, ROWS_MAIN)])

    @pl.when(s == NUM_SUBCORES - 1)
    def _zero_last():
        pltpu.sync_copy(zero_hbm, aggr_sh.at[pl.ds(row_off, ROWS_LAST)])

    pltpu.sync_copy(i1_hbm.at[s], i1_all)
    pltpu.sync_copy(i2_hbm.at[s], i2_all)
    plsc.subcore_barrier()

    def issue_gathers(blk, b):
        pltpu.async_copy(g1_hbm.at[c].at[i1_all.at[blk]], buf_a[b], sem_ga[b])
        pltpu.async_copy(g2_hbm.at[c].at[i2_all.at[blk]], buf_b[b], sem_gb[b])

    def wait_gathers(blk, b):
        pltpu.make_async_copy(g1_hbm.at[c].at[i1_all.at[blk]], buf_a[b],
                              sem_ga[b]).wait()
        pltpu.make_async_copy(g2_hbm.at[c].at[i2_all.at[blk]], buf_b[b],
                              sem_gb[b]).wait()

    def relu_block(b):
        a_buf, b_buf, s_buf = buf_a[b], buf_b[b], buf_s[b]

        @pl.loop(0, E_BLK)
        def _relu_row(r):
            for cb in range(HALF // 16):
                sl = pl.ds(cb * 16, 16)
                s_buf[r, sl] = jnp.maximum(a_buf[r, sl] + b_buf[r, sl], 0.0)

    def issue_scatter(blk, b):
        pltpu.async_copy(buf_s[b], aggr_sh.at[i2_all.at[blk]], sem_sc[b],
                         add=True)

    def wait_scatter(blk, b):
        pltpu.make_async_copy(buf_s[b], aggr_sh.at[i2_all.at[blk]],
                              sem_sc[b]).wait()

    for b in range(2):
        issue_gathers(b, b)
    for b in range(2):
        wait_gathers(b, b)
        relu_block(b)
        issue_gathers(b + 2, b)
        issue_scatter(b, b)

    @pl.loop(1, N_BLKS // 2)
    def _steady(i):
        for b in range(2):
            blk = i * 2 + b
            wait_gathers(blk, b)
            wait_scatter(blk, b)
            relu_block(b)

            @pl.when(blk + 2 < N_BLKS)
            def _refill():
                issue_gathers(blk + 2, b)

            issue_scatter(blk, b)

    for b in range(2):
        wait_scatter(N_BLKS - 2 + b, b)

    plsc.subcore_barrier()

    @pl.when(s < NUM_SUBCORES - 1)
    def _out_main():
        rows = pl.ds(row_off, ROWS_MAIN)
        pltpu.sync_copy(aggr_sh.at[rows], out_hbm.at[c, rows])

    @pl.when(s == NUM_SUBCORES - 1)
    def _out_last():
        rows = pl.ds(row_off, ROWS_LAST)
        pltpu.sync_copy(aggr_sh.at[rows], out_hbm.at[c, rows])


@functools.lru_cache(maxsize=None)
def _make_sc_edge_pass():
    return pl.kernel(
        _sc_edge_body,
        out_type=jax.ShapeDtypeStruct(_SPLIT_SHAPE, jnp.float32),
        mesh=plsc.VectorSubcoreMesh(core_axis_name="c",
                                    subcore_axis_name="s"),
        compiler_params=pltpu.CompilerParams(use_tc_tiling_on_sc=False),
        scratch_types=(
            [pltpu.VMEM((N_BLKS, E_BLK), jnp.int32)] * 2
            + [pltpu.VMEM((E_BLK, HALF), jnp.float32)] * 6
            + [pltpu.VMEM_SHARED((N_NODE, HALF), jnp.float32)]
            + [pltpu.SemaphoreType.DMA] * 6
        ),
    )


def kernel(variables, factors, edge_index, edge_attr, batch_idx,
           Wm_vf, bm_vf, Wc_vf, bc_vf, Wm_fv, bm_fv, Wc_fv, bc_fv):
    del edge_attr, batch_idx
    src = edge_index[0].astype(jnp.int32).reshape(NUM_SUBCORES, N_BLKS, E_BLK)
    dst = edge_index[1].astype(jnp.int32).reshape(NUM_SUBCORES, N_BLKS, E_BLK)
    zeros = jnp.zeros((ROWS_LAST, HALF), jnp.float32)

    def half(w):
        return w[:EMBED], w[EMBED:]

    w1, w2 = half(Wm_vf)
    wc1, wc2 = half(Wc_vf)
    w3, w4 = half(Wm_fv)
    wc3, wc4 = half(Wc_fv)

    bm = bm_vf.reshape(1, EMBED)
    bc = bc_vf.reshape(1, EMBED)
    bm2 = bm_fv.reshape(1, EMBED)
    bc2 = bc_fv.reshape(1, EMBED)

    sc_edge_pass = _make_sc_edge_pass()

    a_proj, b_proj = _tc_pre(factors, variables, w1, w2, bm)
    aggr1 = sc_edge_pass(b_proj, src, a_proj, dst, zeros)
    new_factors, c_proj, d_proj = _tc_mid(
        factors, variables, aggr1, wc1, wc2, bc, w3, w4, bm2)

    aggr2 = sc_edge_pass(d_proj, dst, c_proj, src, zeros)
    new_variables = _tc_post(variables, aggr2, wc3, wc4, bc2)

    return new_variables, new_factors
